# baseline (speedup 1.0000x reference)
import functools

import jax
import jax.numpy as jnp
from jax.experimental import pallas as pl
from jax.experimental.pallas import tpu as pltpu

N = 4096
D = 128
N_RBF = 64
NUM_SPECIES = 8
CELL = 1.0
CUTOFF = 0.12
GAMMA = 0.5 / ((CUTOFF / N_RBF) ** 2)
E_CAP = 131072

_BI = 512
_BJ = 1024


def _mask_body(qi_ref, qj_ref, out_ref):
    i0 = pl.program_id(0) * _BI
    j0 = pl.program_id(1) * _BJ
    dsq = jnp.zeros((_BI, _BJ), jnp.float32)
    for c in range(3):
        d = qj_ref[c, :][None, :] - qi_ref[c, :][:, None]
        d = d + jnp.where(d < -0.5 * CELL, CELL, 0.0) - jnp.where(d >= 0.5 * CELL, CELL, 0.0)
        dsq = dsq + d * d
    ii = i0 + jax.lax.broadcasted_iota(jnp.int32, (_BI, _BJ), 0)
    jj = j0 + jax.lax.broadcasted_iota(jnp.int32, (_BI, _BJ), 1)
    hit = (dsq < CUTOFF * CUTOFF) & (dsq != 0.0) & (ii < jj)
    out_ref[...] = hit


def _pair_mask(qt):
    return pl.pallas_call(
        _mask_body,
        grid=(N // _BI, N // _BJ),
        in_specs=[
            pl.BlockSpec((3, _BI), lambda i, j: (0, i)),
            pl.BlockSpec((3, _BJ), lambda i, j: (0, j)),
        ],
        out_specs=pl.BlockSpec((_BI, _BJ), lambda i, j: (i, j)),
        out_shape=jax.ShapeDtypeStruct((N, N), jnp.bool_),
    )(qt, qt)


def kernel(q, z, emb, W_filt, W_msg, W_upd, W_out):
    qt = q.T
    mask = _pair_mask(qt)
    count = jnp.sum(mask.astype(jnp.int32))
    i, j = jnp.nonzero(mask, size=E_CAP, fill_value=0)
    valid = (jnp.arange(E_CAP) < count).astype(jnp.float32)

    rij = q[j] - q[i]
    offs = -(rij >= 0.5 * CELL).astype(jnp.float32) + (rij < -0.5 * CELL).astype(jnp.float32)
    rij = rij + offs * CELL
    d = jnp.sqrt(jnp.sum(rij**2, axis=-1) + 1e-12)
    mu = jnp.linspace(0.0, CUTOFF, N_RBF, dtype=jnp.float32)
    rbf = jnp.exp(-GAMMA * (d[:, None] - mu[None, :]) ** 2)
    filt = (rbf @ W_filt) * valid[:, None]

    h = emb[z]
    Wc = W_msg @ W_upd
    for _ in range(2):
        pre = jnp.zeros_like(h).at[i].add(h[j] * filt).at[j].add(h[i] * filt)
        h = h + jax.nn.silu(pre @ Wc)
    e_atom = jax.nn.silu(h) @ W_out
    return jnp.sum(e_atom)
